# baseline (speedup 1.0000x reference)
import functools

import jax
import jax.numpy as jnp
import numpy as np
from jax.experimental import pallas as pl

N = 128
M = 8192
STATE_DIM = 3
OBS_DIM = 1024
CTRL_DIM = 7
H = 256

_K1 = 2465931498
_K2 = 255383827
_TINY = float(np.finfo(np.float32).tiny)


def _threefry2x32(x0, x1, k1, k2):
    ks2 = jnp.uint32(k1 ^ k2 ^ 0x1BD11BDA)
    k1 = jnp.uint32(k1)
    k2 = jnp.uint32(k2)
    ks = (k1, k2, ks2)
    rots = ((13, 15, 26, 6), (17, 29, 16, 24))
    x0 = x0 + k1
    x1 = x1 + k2
    for i in range(5):
        for r in rots[i & 1]:
            x0 = x0 + x1
            x1 = (x1 << r) | (x1 >> (32 - r))
            x1 = x1 ^ x0
        x0 = x0 + ks[(i + 1) % 3]
        x1 = x1 + ks[(i + 2) % 3] + jnp.uint32(i + 1)
    return x0, x1


def _sampler_body(logits_t_ref, out_ref, *, n_rows, c_dim, log2_c, log2_nc, unroll):
    mb = pl.program_id(0)
    m_base = mb * 8

    row = jax.lax.broadcasted_iota(jnp.uint32, (8, n_rows), 0)
    col = jax.lax.broadcasted_iota(jnp.uint32, (8, n_rows), 1)

    mlow_mask = (1 << (32 - log2_nc)) - 1
    hi_const = (m_base >> (32 - log2_nc)).astype(jnp.uint32)
    x0_init = jnp.zeros((8, n_rows), jnp.uint32) + hi_const
    lo_base = ((row + (m_base & mlow_mask).astype(jnp.uint32)) << log2_nc) + (
        col << log2_c)

    neg_inf = jnp.float32(-3.0e38)

    def c_body(cb, carry):
        best, bidx = carry
        base = cb * 8
        lt = logits_t_ref[pl.ds(pl.multiple_of(base, 8), 8), :]
        for j in range(8):
            c = base + j
            x1 = lo_base + c.astype(jnp.uint32)
            v0, v1 = _threefry2x32(x0_init, x1, _K1, _K2)
            bits = v0 ^ v1
            fb = (bits >> 9) | jnp.uint32(0x3F800000)
            f = jax.lax.bitcast_convert_type(fb, jnp.float32)
            u = jnp.maximum(f - 1.0, _TINY)
            g = -jnp.log(-jnp.log(u))
            score = g + jnp.broadcast_to(lt[j:j + 1, :], (8, n_rows))
            upd = score > best
            best = jnp.where(upd, score, best)
            bidx = jnp.where(upd, c, bidx)
        return best, bidx

    best0 = jnp.full((8, n_rows), neg_inf, jnp.float32)
    bidx0 = jnp.zeros((8, n_rows), jnp.int32)
    best, bidx = jax.lax.fori_loop(0, c_dim // 8, c_body, (best0, bidx0),
                                   unroll=unroll)
    out_ref[...] = bidx


def _categorical_pallas(logits_t, m_samples, unroll=1):
    c_dim, n_rows = logits_t.shape
    log2_c = int(np.log2(c_dim))
    log2_nc = int(np.log2(n_rows * c_dim))
    body = functools.partial(_sampler_body, n_rows=n_rows, c_dim=c_dim,
                             log2_c=log2_c, log2_nc=log2_nc, unroll=unroll)
    return pl.pallas_call(
        body,
        grid=(m_samples // 8,),
        in_specs=[pl.BlockSpec((c_dim, n_rows), lambda mb: (0, 0))],
        out_specs=pl.BlockSpec((8, n_rows), lambda mb: (mb, 0)),
        out_shape=jax.ShapeDtypeStruct((m_samples, n_rows), jnp.int32),
    )(logits_t)


def _mlp(x, p):
    W1, b1, W2, b2 = p
    return jnp.tanh(x @ W1 + b1) @ W2 + b2


def kernel(states_prev, log_weights_prev, observations, controls, image_params, force_params, weight_params):
    key = jax.random.key(42)
    k_img, k_frc, k_res = jax.random.split(key, 3)

    lw = log_weights_prev - jax.scipy.special.logsumexp(log_weights_prev, axis=1, keepdims=True)

    def branch(params, noise_key):
        dyn, meas = params
        delta = _mlp(controls, dyn)
        noise = 0.01 * jax.random.normal(noise_key, states_prev.shape, jnp.float32)
        states_pred = states_prev + delta[:, None, :] + noise
        target = _mlp(observations, meas)
        err = jnp.sum((states_pred - target[:, None, :]) ** 2, axis=-1)
        lwp = lw - 0.5 * err
        lwp = lwp - jax.scipy.special.logsumexp(lwp, axis=1, keepdims=True)
        est = jnp.sum(jnp.exp(lwp)[..., None] * states_pred, axis=1)
        return est, states_pred, lwp

    img_est, img_sp, img_lwp = branch(image_params, k_img)
    frc_est, frc_sp, frc_lwp = branch(force_params, k_frc)

    wW, wb = weight_params
    log_betas = jax.nn.log_softmax(observations @ wW + wb, axis=-1)
    image_log_beta = log_betas[:, 0:1]
    force_log_beta = log_betas[:, 1:2]
    state_estimates = jnp.exp(image_log_beta) * img_est + jnp.exp(force_log_beta) * frc_est

    states_pred = jnp.concatenate([img_sp, frc_sp], axis=1)
    log_weights_pred = jnp.concatenate(
        [img_lwp + image_log_beta, frc_lwp + force_log_beta], axis=1)

    idx_t = _categorical_pallas(log_weights_pred.T, M)
    idx = idx_t.T
    states = jnp.take_along_axis(states_pred, idx[:, :, None], axis=1)
    log_weights = jnp.full((N, M), -np.log(M), dtype=jnp.float32)
    return state_estimates, states, log_weights
